# baseline (speedup 1.0000x reference)
import functools

import jax
import jax.numpy as jnp
from jax import lax
from jax.experimental import pallas as pl
from jax.experimental.pallas import tpu as pltpu
from jax.experimental.pallas import tpu_sc as plsc

_VOCAB = 100000
_EMBED_DIM = 128
_SEQ_LEN = 200
_BATCH = 4096

_NUM_WORKERS = 32
_B_PER_W = _BATCH // _NUM_WORKERS
_LANES = 16
_INV_L = 1.0 / _SEQ_LEN

_VOCAB_PAD = 100352
_N_WORDS = _VOCAB_PAD // 2


def _tw_body(table_ref, w_ref, out_ref):
    acc = lax.dot_general(
        w_ref[...], table_ref[...],
        dimension_numbers=(((1,), (1,)), ((), ())),
        preferred_element_type=jnp.float32)
    out_ref[...] = (acc[0] * _INV_L).astype(jnp.bfloat16)


def _compute_tw(table, w):
    blk = _VOCAB_PAD // 7
    return pl.pallas_call(
        _tw_body,
        grid=(7,),
        in_specs=[
            pl.BlockSpec((blk, _EMBED_DIM), lambda i: (i, 0)),
            pl.BlockSpec((1, _EMBED_DIM), lambda i: (0, 0)),
        ],
        out_specs=pl.BlockSpec((blk,), lambda i: (i,)),
        out_shape=jax.ShapeDtypeStruct((_VOCAB_PAD,), jnp.bfloat16),
    )(table, w)


def _make_sc_gather_sum():
    mesh = plsc.VectorSubcoreMesh(core_axis_name="c", subcore_axis_name="s")

    @functools.partial(
        pl.kernel,
        mesh=mesh,
        compiler_params=pltpu.CompilerParams(needs_layout_passes=False),
        out_type=jax.ShapeDtypeStruct((_BATCH,), jnp.float32),
        scratch_types=[
            pltpu.VMEM((_N_WORDS,), jnp.int32),
            pltpu.VMEM((_SEQ_LEN, _B_PER_W), jnp.int32),
            pltpu.VMEM((_LANES,), jnp.float32),
            pltpu.VMEM((_B_PER_W,), jnp.float32),
            pltpu.SemaphoreType.DMA,
            pltpu.SemaphoreType.DMA,
            pltpu.SemaphoreType.DMA,
        ],
    )
    def sc_kernel(tw_hbm, x_hbm, bias_hbm, out_hbm, tw_v, idx_v, bias_v,
                  out_v, sem_tw, sem_x, sem_b):
        wid = lax.axis_index("s") * 2 + lax.axis_index("c")
        base = wid * _B_PER_W
        cp_tw = pltpu.async_copy(tw_hbm, tw_v, sem_tw)
        cp_x = pltpu.async_copy(x_hbm.at[:, pl.ds(base, _B_PER_W)], idx_v,
                                sem_x)
        cp_b = pltpu.async_copy(bias_hbm, bias_v, sem_b)
        cp_tw.wait()
        cp_x.wait()
        cp_b.wait()

        nv = _B_PER_W // _LANES
        high_mask = jnp.asarray(-65536, jnp.int32)

        def body(l, accs):
            new = []
            for j in range(nv):
                idx = idx_v[l, pl.ds(j * _LANES, _LANES)]
                word = plsc.load_gather(tw_v, [lax.shift_right_logical(idx, 1)])
                bits = jnp.where(lax.rem(idx, 2) == 1,
                                 lax.bitwise_and(word, high_mask),
                                 lax.shift_left(word, 16))
                new.append(accs[j] + plsc.bitcast(bits, jnp.float32))
            return tuple(new)

        accs = lax.fori_loop(
            0, _SEQ_LEN, body,
            tuple(jnp.zeros((_LANES,), jnp.float32) for _ in range(nv)))
        bv = bias_v[...]
        for j in range(nv):
            out_v[pl.ds(j * _LANES, _LANES)] = accs[j] + bv
        pltpu.sync_copy(out_v, out_hbm.at[pl.ds(base, _B_PER_W)])

    return sc_kernel


_sc_gather_sum = _make_sc_gather_sum()


def kernel(x, table, W, b):
    tw = _compute_tw(table, W)
    tw_words = lax.bitcast_convert_type(
        tw.reshape(_N_WORDS, 2), jnp.int32)
    bias16 = jnp.broadcast_to(b, (_LANES,))
    return _sc_gather_sum(tw_words, x, bias16)


# speedup vs baseline: 1.8817x; 1.8817x over previous
import functools

import jax
import jax.numpy as jnp
from jax import lax
from jax.experimental import pallas as pl
from jax.experimental.pallas import tpu as pltpu
from jax.experimental.pallas import tpu_sc as plsc

_VOCAB = 100000
_EMBED_DIM = 128
_SEQ_LEN = 200
_BATCH = 4096

_NUM_WORKERS = 32
_B_PER_W = _BATCH // _NUM_WORKERS
_LANES = 16
_INV_L = 1.0 / _SEQ_LEN

_VOCAB_PAD = 100352
_N_WORDS = _VOCAB_PAD // 2


_HALF = _VOCAB_PAD // 2
_WBLK = _HALF // 7


def _contract(w_row, t_blk):
    return lax.dot_general(
        w_row, t_blk,
        dimension_numbers=(((1,), (1,)), ((), ())),
        preferred_element_type=jnp.float32)[0]


def _round_bf16_bits(v):
    return lax.bitcast_convert_type(v, jnp.int32) + 32768


def _tw_body(tlo_ref, thi_ref, w_ref, out_ref):
    lo = _contract(w_ref[...], tlo_ref[...]) * _INV_L
    hi = _contract(w_ref[...], thi_ref[...]) * _INV_L
    high_mask = jnp.asarray(-65536, jnp.int32)
    out_ref[...] = lax.bitwise_or(
        lax.shift_right_logical(_round_bf16_bits(lo), 16),
        lax.bitwise_and(_round_bf16_bits(hi), high_mask))


def _compute_tw(table, w):
    return pl.pallas_call(
        _tw_body,
        grid=(7,),
        in_specs=[
            pl.BlockSpec((_WBLK, _EMBED_DIM), lambda i: (i, 0)),
            pl.BlockSpec((_WBLK, _EMBED_DIM), lambda i: (i + 7, 0)),
            pl.BlockSpec((1, _EMBED_DIM), lambda i: (0, 0)),
        ],
        out_specs=pl.BlockSpec((_WBLK,), lambda i: (i,)),
        out_shape=jax.ShapeDtypeStruct((_HALF,), jnp.int32),
    )(table, table, w)


def _make_sc_gather_sum():
    mesh = plsc.VectorSubcoreMesh(core_axis_name="c", subcore_axis_name="s")

    @functools.partial(
        pl.kernel,
        mesh=mesh,
        compiler_params=pltpu.CompilerParams(needs_layout_passes=False),
        out_type=jax.ShapeDtypeStruct((_BATCH,), jnp.float32),
        scratch_types=[
            pltpu.VMEM((_HALF,), jnp.int32),
            pltpu.VMEM((_SEQ_LEN, _B_PER_W), jnp.int32),
            pltpu.VMEM((_LANES,), jnp.float32),
            pltpu.VMEM((_B_PER_W,), jnp.float32),
            pltpu.SemaphoreType.DMA,
            pltpu.SemaphoreType.DMA,
            pltpu.SemaphoreType.DMA,
        ],
    )
    def sc_kernel(tw_hbm, x_hbm, bias_hbm, out_hbm, tw_v, idx_v, bias_v,
                  out_v, sem_tw, sem_x, sem_b):
        wid = lax.axis_index("s") * 2 + lax.axis_index("c")
        base = wid * _B_PER_W
        cp_tw = pltpu.async_copy(tw_hbm, tw_v, sem_tw)
        cp_x = pltpu.async_copy(x_hbm.at[:, pl.ds(base, _B_PER_W)], idx_v,
                                sem_x)
        cp_b = pltpu.async_copy(bias_hbm, bias_v, sem_b)
        cp_tw.wait()
        cp_x.wait()
        cp_b.wait()

        nv = _B_PER_W // _LANES
        high_mask = jnp.asarray(-65536, jnp.int32)

        def body(l, accs):
            new = []
            for j in range(nv):
                idx = idx_v[l, pl.ds(j * _LANES, _LANES)]
                ge = idx >= _HALF
                word = plsc.load_gather(
                    tw_v, [idx - jnp.where(ge, _HALF, 0)])
                bits = jnp.where(ge,
                                 lax.bitwise_and(word, high_mask),
                                 lax.shift_left(word, 16))
                new.append(accs[j] + plsc.bitcast(bits, jnp.float32))
            return tuple(new)

        accs = lax.fori_loop(
            0, _SEQ_LEN, body,
            tuple(jnp.zeros((_LANES,), jnp.float32) for _ in range(nv)))
        bv = bias_v[...]
        for j in range(nv):
            out_v[pl.ds(j * _LANES, _LANES)] = accs[j] + bv
        pltpu.sync_copy(out_v, out_hbm.at[pl.ds(base, _B_PER_W)])

    return sc_kernel


_sc_gather_sum = _make_sc_gather_sum()


def kernel(x, table, W, b):
    tw_words = _compute_tw(table, W)
    bias16 = jnp.broadcast_to(b, (_LANES,))
    return _sc_gather_sum(tw_words, x, bias16)


# speedup vs baseline: 1.9101x; 1.0151x over previous
import functools

import jax
import jax.numpy as jnp
from jax import lax
from jax.experimental import pallas as pl
from jax.experimental.pallas import tpu as pltpu
from jax.experimental.pallas import tpu_sc as plsc

_VOCAB = 100000
_EMBED_DIM = 128
_SEQ_LEN = 200
_BATCH = 4096

_NUM_WORKERS = 32
_B_PER_W = _BATCH // _NUM_WORKERS
_LANES = 16
_INV_L = 1.0 / _SEQ_LEN

_VOCAB_PAD = 100352
_N_WORDS = _VOCAB_PAD // 2


_HALF = _VOCAB_PAD // 2
_WBLK = _HALF // 7


def _contract(w_row, t_blk):
    return lax.dot_general(
        w_row, t_blk,
        dimension_numbers=(((1,), (1,)), ((), ())),
        preferred_element_type=jnp.float32)[0]


def _round_bf16_bits(v):
    return lax.bitcast_convert_type(v, jnp.int32) + 32768


def _tw_body(tlo_ref, thi_ref, w_ref, bias_ref, out_ref, bias16_ref):
    lo = _contract(w_ref[...], tlo_ref[...]) * _INV_L
    hi = _contract(w_ref[...], thi_ref[...]) * _INV_L
    high_mask = jnp.asarray(-65536, jnp.int32)
    out_ref[...] = lax.bitwise_or(
        lax.shift_right_logical(_round_bf16_bits(lo), 16),
        lax.bitwise_and(_round_bf16_bits(hi), high_mask))
    bias16_ref[...] = jnp.full((_LANES,), bias_ref[0], jnp.float32)


def _compute_tw(table, w, bias):
    return pl.pallas_call(
        _tw_body,
        grid=(7,),
        in_specs=[
            pl.BlockSpec((_WBLK, _EMBED_DIM), lambda i: (i, 0)),
            pl.BlockSpec((_WBLK, _EMBED_DIM), lambda i: (i + 7, 0)),
            pl.BlockSpec((1, _EMBED_DIM), lambda i: (0, 0)),
            pl.BlockSpec(memory_space=pltpu.SMEM),
        ],
        out_specs=[
            pl.BlockSpec((_WBLK,), lambda i: (i,)),
            pl.BlockSpec((_LANES,), lambda i: (0,)),
        ],
        out_shape=[
            jax.ShapeDtypeStruct((_HALF,), jnp.int32),
            jax.ShapeDtypeStruct((_LANES,), jnp.float32),
        ],
    )(table, table, w, bias)


def _make_sc_gather_sum():
    mesh = plsc.VectorSubcoreMesh(core_axis_name="c", subcore_axis_name="s")

    @functools.partial(
        pl.kernel,
        mesh=mesh,
        compiler_params=pltpu.CompilerParams(needs_layout_passes=False),
        out_type=jax.ShapeDtypeStruct((_BATCH,), jnp.float32),
        scratch_types=[
            pltpu.VMEM((_HALF,), jnp.int32),
            pltpu.VMEM((_SEQ_LEN, _B_PER_W), jnp.int32),
            pltpu.VMEM((_LANES,), jnp.float32),
            pltpu.VMEM((_B_PER_W,), jnp.float32),
            pltpu.SemaphoreType.DMA,
            pltpu.SemaphoreType.DMA,
            pltpu.SemaphoreType.DMA,
        ],
    )
    def sc_kernel(tw_hbm, x_hbm, bias_hbm, out_hbm, tw_v, idx_v, bias_v,
                  out_v, sem_tw, sem_x, sem_b):
        wid = lax.axis_index("s") * 2 + lax.axis_index("c")
        base = wid * _B_PER_W
        cp_tw = pltpu.async_copy(tw_hbm, tw_v, sem_tw)
        cp_x = pltpu.async_copy(x_hbm.at[:, pl.ds(base, _B_PER_W)], idx_v,
                                sem_x)
        cp_b = pltpu.async_copy(bias_hbm, bias_v, sem_b)
        cp_tw.wait()
        cp_x.wait()
        cp_b.wait()

        nv = _B_PER_W // _LANES
        high_mask = jnp.asarray(-65536, jnp.int32)

        def body(l, accs):
            new = []
            for j in range(nv):
                idx = idx_v[l, pl.ds(j * _LANES, _LANES)]
                ge = idx >= _HALF
                word = plsc.load_gather(
                    tw_v, [idx - jnp.where(ge, _HALF, 0)])
                bits = jnp.where(ge,
                                 lax.bitwise_and(word, high_mask),
                                 lax.shift_left(word, 16))
                new.append(accs[j] + plsc.bitcast(bits, jnp.float32))
            return tuple(new)

        def body2(l2, accs):
            return body(2 * l2 + 1, body(2 * l2, accs))

        accs = lax.fori_loop(
            0, _SEQ_LEN // 2, body2,
            tuple(jnp.zeros((_LANES,), jnp.float32) for _ in range(nv)))
        bv = bias_v[...]
        for j in range(nv):
            out_v[pl.ds(j * _LANES, _LANES)] = accs[j] + bv
        pltpu.sync_copy(out_v, out_hbm.at[pl.ds(base, _B_PER_W)])

    return sc_kernel


_sc_gather_sum = _make_sc_gather_sum()


def kernel(x, table, W, b):
    tw_words, bias16 = _compute_tw(table, W, b)
    return _sc_gather_sum(tw_words, x, bias16)
